# Initial kernel scaffold; baseline (speedup 1.0000x reference)
import jax
import jax.numpy as jnp
from jax.experimental import pallas as pl


def kernel(inputs, Wg, We, be):
    raise NotImplementedError("write your pallas kernel here")


# speedup vs baseline: 2.0192x; 2.0192x over previous
import functools
import math

import jax
import jax.numpy as jnp
from jax.experimental import pallas as pl
from jax.experimental.pallas import tpu as pltpu


def _moe_body(cap, x_ref, wg_ref, we_ref, be_ref, out_ref, cnt_ref):
    i = pl.program_id(0)

    @pl.when(i == 0)
    def _():
        cnt_ref[...] = jnp.zeros_like(cnt_ref)

    x = x_ref[...]
    B = x.shape[0]
    E = wg_ref.shape[1]
    D = x.shape[1]

    logits = jax.lax.dot_general(
        x, wg_ref[...], (((1,), (0,)), ((), ())),
        preferred_element_type=jnp.float32)
    m = jnp.max(logits, axis=1, keepdims=True)
    gate = 1.0 / jnp.sum(jnp.exp(logits - m), axis=1)

    e_iota = jax.lax.broadcasted_iota(jnp.int32, (B, E), 1)
    first = jnp.min(jnp.where(logits == m, e_iota, E), axis=1, keepdims=True)
    mask = (e_iota == first).astype(jnp.float32)

    r = jax.lax.broadcasted_iota(jnp.int32, (B, B), 0)
    c = jax.lax.broadcasted_iota(jnp.int32, (B, B), 1)
    tri = (c <= r).astype(jnp.float32)
    pos_incl = jax.lax.dot_general(
        tri, mask, (((1,), (0,)), ((), ())),
        preferred_element_type=jnp.float32)
    prev = cnt_ref[...]
    pos = pos_incl - 1.0 + prev
    cnt_ref[...] = prev + jnp.sum(mask, axis=0, keepdims=True)

    keep = jnp.sum(jnp.where(pos < cap, mask, 0.0), axis=1)
    scale = (gate * keep)[:, None]

    acc = jnp.zeros((B, D), jnp.float32)
    for e in range(E):
        ye = jax.lax.dot_general(
            x, we_ref[e], (((1,), (0,)), ((), ())),
            preferred_element_type=jnp.float32)
        coef = scale * mask[:, e][:, None]
        acc = acc + coef * (ye + be_ref[e])
    out_ref[...] = acc


def kernel(inputs, Wg, We, be):
    d = inputs.shape[-1]
    E = Wg.shape[1]
    x = inputs.reshape(-1, d)
    T = x.shape[0]
    cap = float(math.ceil(T / E))
    B = 512
    nblocks = T // B

    out = pl.pallas_call(
        functools.partial(_moe_body, cap),
        grid=(nblocks,),
        in_specs=[
            pl.BlockSpec((B, d), lambda i: (i, 0)),
            pl.BlockSpec((d, E), lambda i: (0, 0)),
            pl.BlockSpec((E, d, d), lambda i: (0, 0, 0)),
            pl.BlockSpec((E, d), lambda i: (0, 0)),
        ],
        out_specs=pl.BlockSpec((B, d), lambda i: (i, 0)),
        out_shape=jax.ShapeDtypeStruct((T, d), jnp.float32),
        scratch_shapes=[pltpu.VMEM((1, E), jnp.float32)],
        compiler_params=pltpu.CompilerParams(
            dimension_semantics=("arbitrary",)),
    )(x, Wg, We, be)
    return out.reshape(inputs.shape)


# speedup vs baseline: 6.7061x; 3.3212x over previous
import functools
import math

import jax
import jax.numpy as jnp
from jax.experimental import pallas as pl
from jax.experimental.pallas import tpu as pltpu


def _dot(a, b):
    return jax.lax.dot_general(a, b, (((1,), (0,)), ((), ())),
                               preferred_element_type=jnp.float32)


def _moe_body(cap, x_ref, wg_ref, we_ref, be_ref, u_ref, out_ref, cnt_ref):
    i = pl.program_id(0)

    @pl.when(i == 0)
    def _():
        cnt_ref[...] = jnp.zeros_like(cnt_ref)

    xb = x_ref[...]
    E = wg_ref.shape[0]
    B = xb.shape[1]

    logits = _dot(wg_ref[...], xb)
    m = jnp.max(logits, axis=0, keepdims=True)
    gate = 1.0 / jnp.sum(jnp.exp(logits - m), axis=0, keepdims=True)

    s_iota = jax.lax.broadcasted_iota(jnp.int32, (E, B), 0)
    first = jnp.min(jnp.where(logits == m, s_iota, E), axis=0, keepdims=True)
    mask = (s_iota == first).astype(jnp.float32)

    u = u_ref[...]
    off = cnt_ref[...]
    pos_chunks = []
    for k in range(B // 128):
        pc = _dot(mask[:, k * 128:(k + 1) * 128], u)
        pos_chunks.append(pc + (off - 1.0))
        off = off + pc[:, 127:128]
    cnt_ref[...] = off
    pos = jnp.concatenate(pos_chunks, axis=1)

    keep = jnp.sum(jnp.where(pos < cap, mask, 0.0), axis=0, keepdims=True)
    coef = mask * (gate * keep)

    acc = _dot(be_ref[...], coef)
    for e in range(E):
        acc = acc + _dot(we_ref[e], coef[e:e + 1, :] * xb)
    out_ref[...] = acc


def kernel(inputs, Wg, We, be):
    d = inputs.shape[-1]
    E = Wg.shape[1]
    x = inputs.reshape(-1, d)
    T = x.shape[0]
    cap = float(math.ceil(T / E))
    B = 1024
    nblocks = T // B

    x_T = x.T
    WgT = Wg.T
    WeT = We.transpose(0, 2, 1)
    beT = be.T
    u = jnp.triu(jnp.ones((128, 128), jnp.float32))

    out_T = pl.pallas_call(
        functools.partial(_moe_body, cap),
        grid=(nblocks,),
        in_specs=[
            pl.BlockSpec((d, B), lambda i: (0, i)),
            pl.BlockSpec((E, d), lambda i: (0, 0)),
            pl.BlockSpec((E, d, d), lambda i: (0, 0, 0)),
            pl.BlockSpec((d, E), lambda i: (0, 0)),
            pl.BlockSpec((128, 128), lambda i: (0, 0)),
        ],
        out_specs=pl.BlockSpec((d, B), lambda i: (0, i)),
        out_shape=jax.ShapeDtypeStruct((d, T), jnp.float32),
        scratch_shapes=[pltpu.VMEM((E, 1), jnp.float32)],
        compiler_params=pltpu.CompilerParams(
            dimension_semantics=("arbitrary",)),
    )(x_T, WgT, WeT, beT, u)
    return out_T.T.reshape(inputs.shape)
